# baseline (speedup 1.0000x reference)
import numpy as np
import jax
import jax.numpy as jnp
from jax.experimental import pallas as pl

_RATIOS = np.array([0.5, 1.0, 2.0])
_SCALES = np.array([1.0, 2.0 ** (1.0 / 3.0), 2.0 ** (2.0 / 3.0)])
_LEVELS = [(64, 64, 8, 32), (32, 32, 16, 64), (16, 16, 32, 128), (8, 8, 64, 256)]
_N_ROWS = sum(h * w * 9 for h, w, _, _ in _LEVELS)
_N_PAD = 49152
_PACK_ROWS = _N_PAD // 32
_ROW_OFF = [0, 36864, 46080, 48384]


def _box_sizes(box_size):
    anchors = box_size * np.tile(_SCALES, (2, len(_RATIOS))).T
    areas = anchors[:, 0] * anchors[:, 1]
    anchors[:, 0] = np.sqrt(areas * np.repeat(_RATIOS, len(_SCALES)))
    anchors[:, 1] = anchors[:, 0] / np.repeat(_RATIOS, len(_SCALES))
    return anchors.astype(np.float32)


def _sel_by_level(n, vals, dtype):
    out = jnp.full(n.shape, vals[3], dtype)
    for lvl in (2, 1, 0):
        out = jnp.where(n < _ROW_OFF[lvl + 1], jnp.asarray(vals[lvl], dtype), out)
    return out


def _anchor_kernel(out_xywh, out_xyxy):
    shape = (_PACK_ROWS, 128)
    row = jax.lax.broadcasted_iota(jnp.int32, shape, 0)
    lane = jax.lax.broadcasted_iota(jnp.int32, shape, 1)
    n = 32 * row + jax.lax.shift_right_logical(lane, 2)
    c = jnp.bitwise_and(lane, 3)

    off = _sel_by_level(n, _ROW_OFF, jnp.int32)
    stride_f = _sel_by_level(n, [float(s) for (_, _, s, _) in _LEVELS], jnp.float32)
    mask_w = _sel_by_level(n, [w - 1 for (_, w, _, _) in _LEVELS], jnp.int32)
    lg_w = _sel_by_level(n, [int(np.log2(w)) for (_, w, _, _) in _LEVELS], jnp.int32)
    size_f = _sel_by_level(n, [float(s) for (_, _, _, s) in _LEVELS], jnp.float32)

    q = n - off
    cell = jnp.floor((q.astype(jnp.float32) + 0.5) * (1.0 / 9.0)).astype(jnp.int32)
    a = q - 9 * cell
    wi = jnp.bitwise_and(cell, mask_w)
    hi = jax.lax.shift_right_logical(cell, lg_w)
    x = (wi.astype(jnp.float32) + 0.5) * stride_f
    y = (hi.astype(jnp.float32) + 0.5) * stride_f

    base = _box_sizes(1.0)
    w = jnp.full(shape, float(base[8, 0]), jnp.float32)
    h = jnp.full(shape, float(base[8, 1]), jnp.float32)
    for k in range(7, -1, -1):
        sel = a <= k
        w = jnp.where(sel, float(base[k, 0]), w)
        h = jnp.where(sel, float(base[k, 1]), h)
    w = w * size_f
    h = h * size_f

    c01 = c <= 1
    xy = jnp.where(jnp.bitwise_and(c, 1) == 0, x, y)
    wh = jnp.where(jnp.bitwise_and(c, 1) == 0, w, h)
    out_xywh[:, :] = jnp.where(c01, xy, wh)
    out_xyxy[:, :] = jnp.where(c01, xy - 0.5 * wh, xy + 0.5 * wh)


def kernel(feat_p3, feat_p4, feat_p5, feat_p6):
    del feat_p3, feat_p4, feat_p5, feat_p6
    packed = jax.ShapeDtypeStruct((_PACK_ROWS, 128), jnp.float32)
    a_pk, x_pk = pl.pallas_call(
        _anchor_kernel,
        out_shape=(packed, packed),
    )()
    anchors = a_pk.reshape(_N_PAD, 4)[:_N_ROWS]
    anchors_xyxy = x_pk.reshape(_N_PAD, 4)[:_N_ROWS]
    return anchors, anchors_xyxy


# speedup vs baseline: 1.0062x; 1.0062x over previous
import numpy as np
import jax
import jax.numpy as jnp
from jax.experimental import pallas as pl

_RATIOS = np.array([0.5, 1.0, 2.0])
_SCALES = np.array([1.0, 2.0 ** (1.0 / 3.0), 2.0 ** (2.0 / 3.0)])
_LEVELS = [(64, 64, 8, 32), (32, 32, 16, 64), (16, 16, 32, 128), (8, 8, 64, 256)]
_N_ROWS = sum(h * w * 9 for h, w, _, _ in _LEVELS)
_PACK_ROWS = _N_ROWS * 4 // 128
_ROW_OFF = [0, 36864, 46080, 48384]


def _box_sizes(box_size):
    anchors = box_size * np.tile(_SCALES, (2, len(_RATIOS))).T
    areas = anchors[:, 0] * anchors[:, 1]
    anchors[:, 0] = np.sqrt(areas * np.repeat(_RATIOS, len(_SCALES)))
    anchors[:, 1] = anchors[:, 0] / np.repeat(_RATIOS, len(_SCALES))
    return anchors.astype(np.float32)


def _sel_by_level(n, vals, dtype):
    out = jnp.full(n.shape, vals[3], dtype)
    for lvl in (2, 1, 0):
        out = jnp.where(n < _ROW_OFF[lvl + 1], jnp.asarray(vals[lvl], dtype), out)
    return out


def _anchor_kernel(out_xywh, out_xyxy):
    shape = (_PACK_ROWS, 128)
    row = jax.lax.broadcasted_iota(jnp.int32, shape, 0)
    lane = jax.lax.broadcasted_iota(jnp.int32, shape, 1)
    n = 32 * row + jax.lax.shift_right_logical(lane, 2)
    c = jnp.bitwise_and(lane, 3)

    off = _sel_by_level(n, _ROW_OFF, jnp.int32)
    stride_f = _sel_by_level(n, [float(s) for (_, _, s, _) in _LEVELS], jnp.float32)
    mask_w = _sel_by_level(n, [w - 1 for (_, w, _, _) in _LEVELS], jnp.int32)
    lg_w = _sel_by_level(n, [int(np.log2(w)) for (_, w, _, _) in _LEVELS], jnp.int32)
    size_f = _sel_by_level(n, [float(s) for (_, _, _, s) in _LEVELS], jnp.float32)

    q = n - off
    cell = jnp.floor((q.astype(jnp.float32) + 0.5) * (1.0 / 9.0)).astype(jnp.int32)
    a = q - 9 * cell
    wi = jnp.bitwise_and(cell, mask_w)
    hi = jax.lax.shift_right_logical(cell, lg_w)
    x = (wi.astype(jnp.float32) + 0.5) * stride_f
    y = (hi.astype(jnp.float32) + 0.5) * stride_f

    base = _box_sizes(1.0)
    w = jnp.full(shape, float(base[8, 0]), jnp.float32)
    h = jnp.full(shape, float(base[8, 1]), jnp.float32)
    for k in range(7, -1, -1):
        sel = a <= k
        w = jnp.where(sel, float(base[k, 0]), w)
        h = jnp.where(sel, float(base[k, 1]), h)
    w = w * size_f
    h = h * size_f

    c01 = c <= 1
    xy = jnp.where(jnp.bitwise_and(c, 1) == 0, x, y)
    wh = jnp.where(jnp.bitwise_and(c, 1) == 0, w, h)
    out_xywh[:, :] = jnp.where(c01, xy, wh)
    out_xyxy[:, :] = jnp.where(c01, xy - 0.5 * wh, xy + 0.5 * wh)


def kernel(feat_p3, feat_p4, feat_p5, feat_p6):
    del feat_p3, feat_p4, feat_p5, feat_p6
    packed = jax.ShapeDtypeStruct((_PACK_ROWS, 128), jnp.float32)
    a_pk, x_pk = pl.pallas_call(
        _anchor_kernel,
        out_shape=(packed, packed),
    )()
    anchors = a_pk.reshape(_N_ROWS, 4)
    anchors_xyxy = x_pk.reshape(_N_ROWS, 4)
    return anchors, anchors_xyxy


# speedup vs baseline: 13.4224x; 13.3400x over previous
import numpy as np
import jax
import jax.numpy as jnp
from jax.experimental import pallas as pl

_RATIOS = np.array([0.5, 1.0, 2.0])
_SCALES = np.array([1.0, 2.0 ** (1.0 / 3.0), 2.0 ** (2.0 / 3.0)])
_LEVELS = [(64, 64, 8, 32), (32, 32, 16, 64), (16, 16, 32, 128), (8, 8, 64, 256)]
_N_ROWS = sum(h * w * 9 for h, w, _, _ in _LEVELS)
_N_PAD = 49152
_ROW_OFF = [0, 36864, 46080, 48384]


def _box_sizes(box_size):
    anchors = box_size * np.tile(_SCALES, (2, len(_RATIOS))).T
    areas = anchors[:, 0] * anchors[:, 1]
    anchors[:, 0] = np.sqrt(areas * np.repeat(_RATIOS, len(_SCALES)))
    anchors[:, 1] = anchors[:, 0] / np.repeat(_RATIOS, len(_SCALES))
    return anchors.astype(np.float32)


def _sel_by_level(n, vals, dtype):
    out = jnp.full(n.shape, vals[3], dtype)
    for lvl in (2, 1, 0):
        out = jnp.where(n < _ROW_OFF[lvl + 1], jnp.asarray(vals[lvl], dtype), out)
    return out


def _anchor_kernel(out_cols, out_cols2):
    n = jax.lax.broadcasted_iota(jnp.int32, (1, _N_PAD), 1)
    off = _sel_by_level(n, _ROW_OFF, jnp.int32)
    stride_f = _sel_by_level(n, [float(s) for (_, _, s, _) in _LEVELS], jnp.float32)
    mask_w = _sel_by_level(n, [w - 1 for (_, w, _, _) in _LEVELS], jnp.int32)
    lg_w = _sel_by_level(n, [int(np.log2(w)) for (_, w, _, _) in _LEVELS], jnp.int32)
    size_f = _sel_by_level(n, [float(s) for (_, _, _, s) in _LEVELS], jnp.float32)

    q = n - off
    cell = jnp.floor((q.astype(jnp.float32) + 0.5) * (1.0 / 9.0)).astype(jnp.int32)
    a = q - 9 * cell
    wi = jnp.bitwise_and(cell, mask_w)
    hi = jax.lax.shift_right_logical(cell, lg_w)
    x = (wi.astype(jnp.float32) + 0.5) * stride_f
    y = (hi.astype(jnp.float32) + 0.5) * stride_f

    base = _box_sizes(1.0)
    w = jnp.full(n.shape, float(base[8, 0]), jnp.float32)
    h = jnp.full(n.shape, float(base[8, 1]), jnp.float32)
    for k in range(7, -1, -1):
        sel = a <= k
        w = jnp.where(sel, float(base[k, 0]), w)
        h = jnp.where(sel, float(base[k, 1]), h)
    w = w * size_f
    h = h * size_f

    out_cols[:, :] = jnp.concatenate([x, y, w, h], axis=0)
    out_cols2[:, :] = jnp.concatenate(
        [x - 0.5 * w, y - 0.5 * h, x + 0.5 * w, y + 0.5 * h], axis=0
    )


def kernel(feat_p3, feat_p4, feat_p5, feat_p6):
    del feat_p3, feat_p4, feat_p5, feat_p6
    cols = jax.ShapeDtypeStruct((4, _N_PAD), jnp.float32)
    big0, big1 = pl.pallas_call(
        _anchor_kernel,
        out_shape=(cols, cols),
    )()
    anchors = big0[:, :_N_ROWS].T
    anchors_xyxy = big1[:, :_N_ROWS].T
    return anchors, anchors_xyxy


# speedup vs baseline: 21.7275x; 1.6188x over previous
import numpy as np
import jax
import jax.numpy as jnp
from jax.experimental import pallas as pl

_RATIOS = np.array([0.5, 1.0, 2.0])
_SCALES = np.array([1.0, 2.0 ** (1.0 / 3.0), 2.0 ** (2.0 / 3.0)])
_LEVELS = [(64, 64, 8, 32), (32, 32, 16, 64), (16, 16, 32, 128), (8, 8, 64, 256)]
_N_ROWS = sum(h * w * 9 for h, w, _, _ in _LEVELS)
_N_PAD = 49152
_ROW_OFF = [0, 36864, 46080, 48384]


def _box_sizes(box_size):
    anchors = box_size * np.tile(_SCALES, (2, len(_RATIOS))).T
    areas = anchors[:, 0] * anchors[:, 1]
    anchors[:, 0] = np.sqrt(areas * np.repeat(_RATIOS, len(_SCALES)))
    anchors[:, 1] = anchors[:, 0] / np.repeat(_RATIOS, len(_SCALES))
    return anchors.astype(np.float32)


def _sel_by_level(n, vals, dtype):
    out = jnp.full(n.shape, vals[3], dtype)
    for lvl in (2, 1, 0):
        out = jnp.where(n < _ROW_OFF[lvl + 1], jnp.asarray(vals[lvl], dtype), out)
    return out


def _anchor_kernel(out_cols, out_cols2):
    n = jax.lax.broadcasted_iota(jnp.int32, (1, _N_ROWS), 1)
    off = _sel_by_level(n, _ROW_OFF, jnp.int32)
    stride_f = _sel_by_level(n, [float(s) for (_, _, s, _) in _LEVELS], jnp.float32)
    mask_w = _sel_by_level(n, [w - 1 for (_, w, _, _) in _LEVELS], jnp.int32)
    lg_w = _sel_by_level(n, [int(np.log2(w)) for (_, w, _, _) in _LEVELS], jnp.int32)
    size_f = _sel_by_level(n, [float(s) for (_, _, _, s) in _LEVELS], jnp.float32)

    q = n - off
    cell = jnp.floor((q.astype(jnp.float32) + 0.5) * (1.0 / 9.0)).astype(jnp.int32)
    a = q - 9 * cell
    wi = jnp.bitwise_and(cell, mask_w)
    hi = jax.lax.shift_right_logical(cell, lg_w)
    x = (wi.astype(jnp.float32) + 0.5) * stride_f
    y = (hi.astype(jnp.float32) + 0.5) * stride_f

    base = _box_sizes(1.0)
    w = jnp.full(n.shape, float(base[8, 0]), jnp.float32)
    h = jnp.full(n.shape, float(base[8, 1]), jnp.float32)
    for k in range(7, -1, -1):
        sel = a <= k
        w = jnp.where(sel, float(base[k, 0]), w)
        h = jnp.where(sel, float(base[k, 1]), h)
    w = w * size_f
    h = h * size_f

    out_cols[:, :] = jnp.concatenate([x, y, w, h], axis=0)
    out_cols2[:, :] = jnp.concatenate(
        [x - 0.5 * w, y - 0.5 * h, x + 0.5 * w, y + 0.5 * h], axis=0
    )


def kernel(feat_p3, feat_p4, feat_p5, feat_p6):
    del feat_p3, feat_p4, feat_p5, feat_p6
    cols = jax.ShapeDtypeStruct((4, _N_ROWS), jnp.float32)
    big0, big1 = pl.pallas_call(
        _anchor_kernel,
        out_shape=(cols, cols),
    )()
    return big0.T, big1.T
